# baseline (speedup 1.0000x reference)
import functools

import jax
import jax.numpy as jnp
from jax.experimental import pallas as pl
from jax.experimental.pallas import tpu as pltpu

_BR = 512
_NBUF = 2
_CSPLIT = 4


def _col_chunks(m):
    chunk = (m // (_CSPLIT * 128)) * 128
    offs = [c * chunk for c in range(_CSPLIT)]
    widths = [chunk] * (_CSPLIT - 1) + [m - chunk * (_CSPLIT - 1)]
    return list(zip(offs, widths))


def _copy_block(src_ref, row_start, nrows, buf, sems, k, m, do_wait):
    for c, (off, w) in enumerate(_col_chunks(m)):
        copy = pltpu.make_async_copy(
            src_ref.at[pl.ds(row_start, nrows), pl.ds(off, w)],
            buf.at[0:nrows, pl.ds(off, w)],
            sems.at[k, c],
        )
        copy.wait() if do_wait else copy.start()


def _block_dma(j, e_ref, a_ref, bufs, sems, g, m, do_wait):
    slot = jax.lax.rem(j, _NBUF)
    tail = m - (g - 1) * _BR
    is_e = j < g
    jl = jnp.where(is_e, j, j - g)
    not_tail = jl < g - 1

    for k in range(_NBUF):
        on = slot == k
        buf = bufs[k]

        @pl.when(on & is_e & not_tail)
        def _(buf=buf, k=k):
            _copy_block(e_ref, jl * _BR, _BR, buf, sems, k, m, do_wait)

        @pl.when(on & is_e & ~not_tail)
        def _(buf=buf, k=k):
            _copy_block(e_ref, (g - 1) * _BR, tail, buf, sems, k, m, do_wait)

        @pl.when(on & ~is_e & not_tail)
        def _(buf=buf, k=k):
            _copy_block(a_ref, jl * _BR, _BR, buf, sems, k, m, do_wait)

        @pl.when(on & ~is_e & ~not_tail)
        def _(buf=buf, k=k):
            _copy_block(a_ref, (g - 1) * _BR, tail, buf, sems, k, m, do_wait)


def _fused_kernel(
    e_ref, a_ref, x_ref, y_ref, wfc_ref, bfc_ref, w1_ref, b1_ref, w2_ref,
    b2_ref, o_ref, xe_ref, *scratch, g, m
):
    bufs = scratch[:_NBUF]
    sems, s1_scr, s2_scr = scratch[_NBUF:]
    i = pl.program_id(0)

    @pl.when(i == 0)
    def _startup():
        for j in range(_NBUF - 1):
            _copy_block(e_ref, j * _BR, _BR, bufs[j], sems, j, m, False)
        y_new = (
            jnp.dot(y_ref[:], wfc_ref[:], preferred_element_type=jnp.float32)
            + bfc_ref[:]
        )
        n_nodes = x_ref.shape[0]
        s1_scr[0:n_nodes, :] = jnp.dot(
            x_ref[:], w1_ref[:], preferred_element_type=jnp.float32
        )
        s1_scr[n_nodes:, :] = jnp.dot(
            y_new, w1_ref[:], preferred_element_type=jnp.float32
        )

    nxt = i + _NBUF - 1

    @pl.when(nxt < 2 * g)
    def _prefetch():
        _block_dma(nxt, e_ref, a_ref, bufs, sems, g, m, do_wait=False)

    _block_dma(i, e_ref, a_ref, bufs, sems, g, m, do_wait=True)

    slot = jax.lax.rem(i, _NBUF)

    def _compute(blk_ref):
        @pl.when(i < g)
        def _phase1():
            acc = jnp.dot(
                blk_ref[:], s1_scr[:], preferred_element_type=jnp.float32
            )
            xe = jnp.maximum(acc + b1_ref[:], 0.0)
            xe_ref[:] = xe
            s2_scr[pl.ds(i * _BR, _BR), :] = jnp.dot(
                xe, w2_ref[:], preferred_element_type=jnp.float32
            )

        @pl.when(i >= g)
        def _phase2():
            acc = jnp.dot(
                blk_ref[:], s2_scr[0:m, :], preferred_element_type=jnp.float32
            )
            o_ref[:] = jax.nn.sigmoid(acc + b2_ref[:])

    for k in range(_NBUF):
        @pl.when(slot == k)
        def _(k=k):
            _compute(bufs[k])


def kernel(Y_embedding, X, E_tilde, A_tilde, W_fc, b_fc, W1, b1, W2, b2):
    m = E_tilde.shape[0]
    n = X.shape[0]
    nfeat = X.shape[1]
    nhid = W1.shape[1]
    nclass = W2.shape[1]
    nhigh = Y_embedding.shape[1]
    l = Y_embedding.shape[0]
    f32 = jnp.float32

    bfc2 = b_fc.reshape(1, nfeat)
    b1_2 = b1.reshape(1, nhid)
    b2_2 = b2.reshape(1, nclass)

    g = pl.cdiv(m, _BR)

    const = lambda i: (0, 0)
    hbm = pl.BlockSpec(memory_space=pltpu.MemorySpace.HBM)
    body = functools.partial(_fused_kernel, g=g, m=m)

    output, x_embedding = pl.pallas_call(
        body,
        grid=(2 * g,),
        in_specs=[
            hbm,
            hbm,
            pl.BlockSpec((n, nfeat), const),
            pl.BlockSpec((l, nhigh), const),
            pl.BlockSpec((nhigh, nfeat), const),
            pl.BlockSpec((1, nfeat), const),
            pl.BlockSpec((nfeat, nhid), const),
            pl.BlockSpec((1, nhid), const),
            pl.BlockSpec((nhid, nclass), const),
            pl.BlockSpec((1, nclass), const),
        ],
        out_specs=[
            pl.BlockSpec((_BR, nclass), lambda i: (jnp.maximum(i - g, 0), 0)),
            pl.BlockSpec((_BR, nhid), lambda i: (jnp.minimum(i, g - 1), 0)),
        ],
        out_shape=[
            jax.ShapeDtypeStruct((m, nclass), f32),
            jax.ShapeDtypeStruct((m, nhid), f32),
        ],
        scratch_shapes=(
            [pltpu.VMEM((_BR, m), f32) for _ in range(_NBUF)]
            + [
                pltpu.SemaphoreType.DMA((_NBUF, _CSPLIT)),
                pltpu.VMEM((m, nhid), f32),
                pltpu.VMEM((g * _BR, nclass), f32),
            ]
        ),
        compiler_params=pltpu.CompilerParams(
            dimension_semantics=("arbitrary",)
        ),
    )(E_tilde, A_tilde, X, Y_embedding, W_fc, bfc2, W1, b1_2, W2, b2_2)

    return (output, x_embedding)


# speedup vs baseline: 1.0247x; 1.0247x over previous
import functools

import jax
import jax.numpy as jnp
from jax.experimental import pallas as pl
from jax.experimental.pallas import tpu as pltpu

_BK = 256


def _prep_kernel(x_ref, y_ref, wfc_ref, bfc_ref, w1_ref, s1t_ref, s1_scr):
    y_new = (
        jnp.dot(y_ref[:], wfc_ref[:], preferred_element_type=jnp.float32)
        + bfc_ref[:]
    )
    n_nodes = x_ref.shape[0]
    s1_scr[0:n_nodes, :] = jnp.dot(
        x_ref[:], w1_ref[:], preferred_element_type=jnp.float32
    )
    s1_scr[n_nodes:, :] = jnp.dot(
        y_new, w1_ref[:], preferred_element_type=jnp.float32
    )
    s1t_ref[:] = s1_scr[:].T


def _main_kernel(
    e_ref, a_ref, s1t_ref, b1_ref, w2_ref, b2_ref, o_ref, xe_ref, *, g, m
):
    i = pl.program_id(0)

    xe = jnp.maximum(
        jax.lax.dot_general(
            e_ref[:], s1t_ref[:], (((1,), (1,)), ((), ())),
            preferred_element_type=jnp.float32,
        )
        + b1_ref[:],
        0.0,
    )
    xe_ref[:] = xe

    s2_blk = jnp.dot(xe, w2_ref[:], preferred_element_type=jnp.float32)
    valid = jnp.where(i == g - 1, m - (g - 1) * _BK, _BK)
    row_ids = jax.lax.broadcasted_iota(jnp.int32, s2_blk.shape, 0)
    s2_blk = jnp.where(row_ids < valid, s2_blk, 0.0)

    a_blk = a_ref[:]

    def contrib(a):
        return jax.lax.dot_general(
            s2_blk, a, (((0,), (1,)), ((), ())),
            preferred_element_type=jnp.float32,
        )

    @pl.when(i < g - 1)
    def _acc_full():
        c = contrib(a_blk)

        @pl.when(i == 0)
        def _():
            o_ref[:] = c

        @pl.when(i > 0)
        def _():
            o_ref[:] = o_ref[:] + c

    @pl.when(i == g - 1)
    def _acc_tail():
        col_ids = jax.lax.broadcasted_iota(jnp.int32, a_blk.shape, 1)
        a_masked = jnp.where(col_ids < valid, a_blk, 0.0)
        c = contrib(a_masked)
        total = c if g == 1 else o_ref[:] + c
        o_ref[:] = jax.nn.sigmoid(total + b2_ref[:])


def kernel(Y_embedding, X, E_tilde, A_tilde, W_fc, b_fc, W1, b1, W2, b2):
    m = E_tilde.shape[0]
    n = X.shape[0]
    nfeat = X.shape[1]
    nhid = W1.shape[1]
    nclass = W2.shape[1]
    nhigh = Y_embedding.shape[1]
    l = Y_embedding.shape[0]
    f32 = jnp.float32

    bfc2 = b_fc.reshape(1, nfeat)
    b1_2 = b1.reshape(1, nhid)
    b2_2 = b2.reshape(nclass, 1)

    g = pl.cdiv(m, _BK)
    const = lambda i: (0, 0)

    s1t = pl.pallas_call(
        _prep_kernel,
        out_shape=jax.ShapeDtypeStruct((nhid, m), f32),
        scratch_shapes=[pltpu.VMEM((m, nhid), f32)],
    )(X, Y_embedding, W_fc, bfc2, W1)

    body = functools.partial(_main_kernel, g=g, m=m)

    output_t, x_embedding = pl.pallas_call(
        body,
        grid=(g,),
        in_specs=[
            pl.BlockSpec((_BK, m), lambda i: (i, 0)),
            pl.BlockSpec((m, _BK), lambda i: (0, i)),
            pl.BlockSpec((nhid, m), const),
            pl.BlockSpec((1, nhid), const),
            pl.BlockSpec((nhid, nclass), const),
            pl.BlockSpec((nclass, 1), const),
        ],
        out_specs=[
            pl.BlockSpec((nclass, m), const),
            pl.BlockSpec((_BK, nhid), lambda i: (i, 0)),
        ],
        out_shape=[
            jax.ShapeDtypeStruct((nclass, m), f32),
            jax.ShapeDtypeStruct((m, nhid), f32),
        ],
        compiler_params=pltpu.CompilerParams(
            dimension_semantics=("arbitrary",)
        ),
    )(E_tilde, A_tilde, s1t, b1_2, W2, b2_2)

    return (output_t.T, x_embedding)


# speedup vs baseline: 1.0452x; 1.0200x over previous
import functools

import jax
import jax.numpy as jnp
from jax.experimental import pallas as pl
from jax.experimental.pallas import tpu as pltpu

_BK = 256


def _main_kernel(
    e_ref, a_ref, x_ref, y_ref, wfc_ref, bfc_ref, w1_ref, b1_ref, w2_ref,
    b2_ref, o_ref, xe_ref, s1_scr, *, g, m
):
    i = pl.program_id(0)

    @pl.when(i == 0)
    def _prep():
        y_new = (
            jnp.dot(y_ref[:], wfc_ref[:], preferred_element_type=jnp.float32)
            + bfc_ref[:]
        )
        n_nodes = x_ref.shape[0]
        s1_scr[0:n_nodes, :] = jnp.dot(
            x_ref[:], w1_ref[:], preferred_element_type=jnp.float32
        )
        s1_scr[n_nodes:, :] = jnp.dot(
            y_new, w1_ref[:], preferred_element_type=jnp.float32
        )

    xe = jnp.maximum(
        jnp.dot(e_ref[:], s1_scr[:], preferred_element_type=jnp.float32)
        + b1_ref[:],
        0.0,
    )
    xe_ref[:] = xe

    s2_blk = jnp.dot(xe, w2_ref[:], preferred_element_type=jnp.float32)

    def contrib(s2, a):
        return jax.lax.dot_general(
            s2, a, (((0,), (1,)), ((), ())),
            preferred_element_type=jnp.float32,
        )

    @pl.when(i < g - 1)
    def _acc_full():
        c = contrib(s2_blk, a_ref[:])

        @pl.when(i == 0)
        def _():
            o_ref[:] = c

        @pl.when(i > 0)
        def _():
            o_ref[:] = o_ref[:] + c

    @pl.when(i == g - 1)
    def _acc_tail():
        t = m - (g - 1) * _BK
        c = contrib(s2_blk[0:t, :], a_ref[:, 0:t])
        total = c if g == 1 else o_ref[:] + c
        o_ref[:] = jax.nn.sigmoid(total + b2_ref[:])


def kernel(Y_embedding, X, E_tilde, A_tilde, W_fc, b_fc, W1, b1, W2, b2):
    m = E_tilde.shape[0]
    n = X.shape[0]
    nfeat = X.shape[1]
    nhid = W1.shape[1]
    nclass = W2.shape[1]
    nhigh = Y_embedding.shape[1]
    l = Y_embedding.shape[0]
    f32 = jnp.float32

    bfc2 = b_fc.reshape(1, nfeat)
    b1_2 = b1.reshape(1, nhid)
    b2t = b2.reshape(nclass, 1)

    g = pl.cdiv(m, _BK)
    const = lambda i: (0, 0)
    body = functools.partial(_main_kernel, g=g, m=m)

    output_t, x_embedding = pl.pallas_call(
        body,
        grid=(g,),
        in_specs=[
            pl.BlockSpec((_BK, m), lambda i: (i, 0)),
            pl.BlockSpec((m, _BK), lambda i: (0, i)),
            pl.BlockSpec((n, nfeat), const),
            pl.BlockSpec((l, nhigh), const),
            pl.BlockSpec((nhigh, nfeat), const),
            pl.BlockSpec((1, nfeat), const),
            pl.BlockSpec((nfeat, nhid), const),
            pl.BlockSpec((1, nhid), const),
            pl.BlockSpec((nhid, nclass), const),
            pl.BlockSpec((nclass, 1), const),
        ],
        out_specs=[
            pl.BlockSpec((nclass, m), const),
            pl.BlockSpec((_BK, nhid), lambda i: (i, 0)),
        ],
        out_shape=[
            jax.ShapeDtypeStruct((nclass, m), f32),
            jax.ShapeDtypeStruct((m, nhid), f32),
        ],
        scratch_shapes=[
            pltpu.VMEM((m, nhid), f32),
        ],
        compiler_params=pltpu.CompilerParams(
            dimension_semantics=("arbitrary",)
        ),
    )(E_tilde, A_tilde, X, Y_embedding, W_fc, bfc2, W1, b1_2, W2, b2t)

    return (output_t.T, x_embedding)
